# baseline (speedup 1.0000x reference)
import functools

import jax
import jax.numpy as jnp
from jax import lax
from jax.experimental import pallas as pl
from jax.experimental.pallas import tpu as pltpu
from jax.experimental.pallas import tpu_sc as plsc

NC = 2
NS = 16
LANES = 16
CHUNK = 128
NIDX = 4


def _sc_mesh():
    return plsc.VectorSubcoreMesh(core_axis_name="c", subcore_axis_name="s")


def _sc_degree(dst_t, n_pad, rows_per_tile):
    n_chunks = dst_t.shape[1]

    @functools.partial(
        pl.kernel,
        out_type=jax.ShapeDtypeStruct((NC, n_pad), jnp.float32),
        mesh=_sc_mesh(),
        scratch_types=[
            pltpu.VMEM((n_chunks, CHUNK), jnp.int32),
            pltpu.VMEM((CHUNK,), jnp.float32),
            pltpu.VMEM((rows_per_tile,), jnp.float32),
            pltpu.VMEM_SHARED((n_pad,), jnp.float32),
            pltpu.SemaphoreType.DMA,
        ],
    )
    def k(dst_hbm, out_hbm, idx_all, ones_v, zeros_v, deg_shared, sem):
        c = lax.axis_index("c")
        s = lax.axis_index("s")
        w = c * NS + s
        row0 = pl.multiple_of(s * rows_per_tile, CHUNK)

        pltpu.sync_copy(dst_hbm.at[w], idx_all)

        @pl.loop(0, rows_per_tile // LANES)
        def _zero(i):
            zeros_v[pl.ds(i * LANES, LANES)] = jnp.zeros((LANES,), jnp.float32)

        pltpu.sync_copy(zeros_v, deg_shared.at[pl.ds(row0, rows_per_tile)])
        for j in range(CHUNK // LANES):
            ones_v[pl.ds(j * LANES, LANES)] = jnp.ones((LANES,), jnp.float32)
        plsc.subcore_barrier()

        @pl.loop(0, n_chunks)
        def _fire(i):
            pltpu.async_copy(ones_v, deg_shared.at[idx_all.at[i]], sem, add=True)

        @pl.loop(0, n_chunks)
        def _drain(i):
            pltpu.make_async_copy(ones_v, deg_shared.at[idx_all.at[i]], sem).wait()

        plsc.subcore_barrier()
        pltpu.sync_copy(
            deg_shared.at[pl.ds(row0, rows_per_tile)],
            out_hbm.at[c, pl.ds(row0, rows_per_tile)],
        )

    return k(dst_t)


def _sc_aggregate(g, src_t, dst_t, n_pad, rows_per_tile):
    n, d = g.shape
    n_chunks = src_t.shape[1]
    drain_chunks = rows_per_tile // CHUNK

    @functools.partial(
        pl.kernel,
        out_type=jax.ShapeDtypeStruct((NC, n_pad, d), jnp.float32),
        mesh=_sc_mesh(),
        scratch_types=[
            pltpu.VMEM((NIDX, 2, CHUNK), jnp.int32),
            pltpu.VMEM((2, CHUNK, d), jnp.float32),
            pltpu.VMEM_SHARED((n_pad, d), jnp.float32),
        ]
        + [pltpu.SemaphoreType.DMA] * (NIDX + 4),
    )
    def k(g_hbm, src_hbm, dst_hbm, out_hbm, sd, rows, acc_shared, *sems):
        isem = sems[:NIDX]
        gsem = sems[NIDX:NIDX + 2]
        ssem = sems[NIDX + 2:]
        c = lax.axis_index("c")
        s = lax.axis_index("s")
        w = c * NS + s
        row0 = pl.multiple_of(s * rows_per_tile, CHUNK)

        def i_start(i, q):
            pltpu.async_copy(src_hbm.at[w, i], sd.at[q, 0], isem[q])
            pltpu.async_copy(dst_hbm.at[w, i], sd.at[q, 1], isem[q])

        def i_wait(i, q):
            pltpu.make_async_copy(src_hbm.at[w, i], sd.at[q, 0], isem[q]).wait()
            pltpu.make_async_copy(dst_hbm.at[w, i], sd.at[q, 1], isem[q]).wait()

        def g_start(q, b):
            pltpu.async_copy(g_hbm.at[sd.at[q, 0]], rows.at[b], gsem[b])

        def g_wait(q, b):
            pltpu.make_async_copy(g_hbm.at[sd.at[q, 0]], rows.at[b], gsem[b]).wait()

        def s_start(q, b):
            pltpu.async_copy(rows.at[b], acc_shared.at[sd.at[q, 1]], ssem[b], add=True)

        def s_wait(q, b):
            pltpu.make_async_copy(rows.at[b], acc_shared.at[sd.at[q, 1]], ssem[b]).wait()

        @pl.loop(0, CHUNK)
        def _zrows(i):
            for j in range(d // LANES):
                rows[0, i, pl.ds(j * LANES, LANES)] = jnp.zeros((LANES,), jnp.float32)

        @pl.loop(0, drain_chunks)
        def _zacc(kk):
            r = pl.multiple_of(row0 + kk * CHUNK, CHUNK)
            pltpu.sync_copy(rows.at[0], acc_shared.at[pl.ds(r, CHUNK)])

        plsc.subcore_barrier()

        for q in range(2):
            i_start(q, q)
        for q in range(2):
            i_wait(q, q)
            g_start(q, q)

        @pl.loop(0, n_chunks // 4)
        def _pipe(t):
            for k in range(4):
                i = t * 4 + k
                q = k
                qn = (k + 2) % 4
                bslot = k % 2

                @pl.when(i + 2 < n_chunks)
                def _pre():
                    i_start(i + 2, qn)

                g_wait(q, bslot)
                s_start(q, bslot)

                @pl.when(i + 2 < n_chunks)
                def _nxt():
                    s_wait(q, bslot)
                    i_wait(i + 2, qn)
                    g_start(qn, bslot)

        s_wait(2, 0)
        s_wait(3, 1)

        plsc.subcore_barrier()

        @pl.loop(0, drain_chunks)
        def _drain(kk):
            r = pl.multiple_of(row0 + kk * CHUNK, CHUNK)
            pltpu.sync_copy(
                acc_shared.at[pl.ds(r, CHUNK)], out_hbm.at[c, pl.ds(r, CHUNK)]
            )

    return k(g, src_t, dst_t)


def _tc_matmul(x, w):
    n, d_in = x.shape
    d_out = w.shape[1]
    blk = 1000

    def body(x_ref, w_ref, h_ref):
        h_ref[...] = jnp.dot(x_ref[...], w_ref[...],
                             preferred_element_type=jnp.float32)

    return pl.pallas_call(
        body,
        grid=(n // blk,),
        in_specs=[
            pl.BlockSpec((blk, d_in), lambda i: (i, 0)),
            pl.BlockSpec((d_in, d_out), lambda i: (0, 0)),
        ],
        out_specs=pl.BlockSpec((blk, d_out), lambda i: (i, 0)),
        out_shape=jax.ShapeDtypeStruct((n, d_out), jnp.float32),
    )(x, w)


def _tc_scale(h, deg2):
    n, d = h.shape
    blk = 1000

    def body(h_ref, deg_ref, g_ref, dis_ref):
        deg = deg_ref[:, 0:1] + deg_ref[:, 1:2] + 1.0
        dis = lax.rsqrt(deg)
        g_ref[...] = h_ref[...] * dis
        dis_ref[...] = dis

    return pl.pallas_call(
        body,
        grid=(n // blk,),
        in_specs=[
            pl.BlockSpec((blk, d), lambda i: (i, 0)),
            pl.BlockSpec((blk, 2), lambda i: (i, 0)),
        ],
        out_specs=[
            pl.BlockSpec((blk, d), lambda i: (i, 0)),
            pl.BlockSpec((blk, 1), lambda i: (i, 0)),
        ],
        out_shape=[
            jax.ShapeDtypeStruct((n, d), jnp.float32),
            jax.ShapeDtypeStruct((n, 1), jnp.float32),
        ],
    )(h, deg2)


def _tc_combine(acc, g, dis, b2, n):
    d = g.shape[1]
    blk = 1000

    def body(acc_ref, g_ref, dis_ref, b_ref, out_ref):
        total = acc_ref[0] + acc_ref[1] + g_ref[...]
        out_ref[...] = dis_ref[...] * total + b_ref[...]

    return pl.pallas_call(
        body,
        grid=(n // blk,),
        in_specs=[
            pl.BlockSpec((2, blk, d), lambda i: (0, i, 0)),
            pl.BlockSpec((blk, d), lambda i: (i, 0)),
            pl.BlockSpec((blk, 1), lambda i: (i, 0)),
            pl.BlockSpec((1, d), lambda i: (0, 0)),
        ],
        out_specs=pl.BlockSpec((blk, d), lambda i: (i, 0)),
        out_shape=jax.ShapeDtypeStruct((n, d), jnp.float32),
    )(acc, g, dis, b2)


def kernel(x, edge_index, W, b):
    n, d_in = x.shape
    d_out = W.shape[1]
    e = edge_index.shape[1]
    nw = NC * NS

    rows_per_tile = pl.cdiv(n + 1, NS * CHUNK) * CHUNK
    n_pad = NS * rows_per_tile

    src = edge_index[0]
    dst = edge_index[1]

    e_tile = pl.cdiv(e, nw * 2 * NIDX * CHUNK) * 2 * NIDX * CHUNK
    pad = nw * e_tile - e
    zsrc = jnp.arange(pad, dtype=jnp.int32) % n
    zdst = n + jnp.arange(pad, dtype=jnp.int32) % (n_pad - n)
    nch = e_tile // CHUNK
    src_t = jnp.concatenate([src, zsrc]).reshape(nw, nch, CHUNK)
    dst_t = jnp.concatenate([dst, zdst]).reshape(nw, nch, CHUNK)

    h = _tc_matmul(x, W)
    degs = _sc_degree(dst_t, n_pad, rows_per_tile)
    deg2 = degs.T[:n]
    g, dis = _tc_scale(h, deg2)

    acc = _sc_aggregate(g, src_t, dst_t, n_pad, rows_per_tile)
    out = _tc_combine(acc, g, dis, b.reshape(1, d_out), n)
    return out


# speedup vs baseline: 1.0998x; 1.0998x over previous
import functools

import jax
import jax.numpy as jnp
from jax import lax
from jax.experimental import pallas as pl
from jax.experimental.pallas import tpu as pltpu
from jax.experimental.pallas import tpu_sc as plsc

NC = 2
NS = 16
LANES = 16
CHUNK = 128
NIDX = 4


def _sc_mesh():
    return plsc.VectorSubcoreMesh(core_axis_name="c", subcore_axis_name="s")


def _sc_degree(edges, n_pad, rows_per_tile, n_chunks):
    nw = NC * NS

    @functools.partial(
        pl.kernel,
        out_type=jax.ShapeDtypeStruct((NC, n_pad), jnp.float32),
        mesh=_sc_mesh(),
        scratch_types=[
            pltpu.VMEM((n_chunks, CHUNK), jnp.int32),
            pltpu.VMEM((CHUNK,), jnp.float32),
            pltpu.VMEM((rows_per_tile,), jnp.float32),
            pltpu.VMEM_SHARED((n_pad,), jnp.float32),
            pltpu.SemaphoreType.DMA,
            pltpu.SemaphoreType.DMA,
        ],
    )
    def k(edges_hbm, out_hbm, idx_all, ones_v, zeros_v, deg_shared, psem, sem):
        c = lax.axis_index("c")
        s = lax.axis_index("s")
        w = c * NS + s
        row0 = pl.multiple_of(s * rows_per_tile, CHUNK)

        @pl.loop(0, n_chunks)
        def _pfire(i):
            col = pl.multiple_of((i * nw + w) * CHUNK, CHUNK)
            pltpu.async_copy(edges_hbm.at[1, pl.ds(col, CHUNK)], idx_all.at[i], psem)

        @pl.loop(0, rows_per_tile // LANES)
        def _zero(i):
            zeros_v[pl.ds(i * LANES, LANES)] = jnp.zeros((LANES,), jnp.float32)

        pltpu.sync_copy(zeros_v, deg_shared.at[pl.ds(row0, rows_per_tile)])
        for j in range(CHUNK // LANES):
            ones_v[pl.ds(j * LANES, LANES)] = jnp.ones((LANES,), jnp.float32)

        @pl.loop(0, n_chunks)
        def _pdrain(i):
            col = pl.multiple_of((i * nw + w) * CHUNK, CHUNK)
            pltpu.make_async_copy(
                edges_hbm.at[1, pl.ds(col, CHUNK)], idx_all.at[i], psem
            ).wait()

        plsc.subcore_barrier()

        @pl.loop(0, n_chunks)
        def _fire(i):
            pltpu.async_copy(ones_v, deg_shared.at[idx_all.at[i]], sem, add=True)

        @pl.loop(0, n_chunks)
        def _drain(i):
            pltpu.make_async_copy(ones_v, deg_shared.at[idx_all.at[i]], sem).wait()

        plsc.subcore_barrier()
        pltpu.sync_copy(
            deg_shared.at[pl.ds(row0, rows_per_tile)],
            out_hbm.at[c, pl.ds(row0, rows_per_tile)],
        )

    return k(edges)


def _sc_aggregate(g, edges, n_pad, rows_per_tile, n_chunks):
    n, d = g.shape
    nw = NC * NS
    drain_chunks = rows_per_tile // CHUNK

    @functools.partial(
        pl.kernel,
        out_type=jax.ShapeDtypeStruct((NC, n_pad, d), jnp.float32),
        mesh=_sc_mesh(),
        scratch_types=[
            pltpu.VMEM((NIDX, 2, CHUNK), jnp.int32),
            pltpu.VMEM((2, CHUNK, d), jnp.float32),
            pltpu.VMEM_SHARED((n_pad, d), jnp.float32),
        ]
        + [pltpu.SemaphoreType.DMA] * (NIDX + 4),
    )
    def k(g_hbm, edges_hbm, out_hbm, sd, rows, acc_shared, *sems):
        isem = sems[:NIDX]
        gsem = sems[NIDX:NIDX + 2]
        ssem = sems[NIDX + 2:]
        c = lax.axis_index("c")
        s = lax.axis_index("s")
        w = c * NS + s
        row0 = pl.multiple_of(s * rows_per_tile, CHUNK)

        def i_start(i, q):
            col = pl.multiple_of((i * nw + w) * CHUNK, CHUNK)
            pltpu.async_copy(edges_hbm.at[0, pl.ds(col, CHUNK)], sd.at[q, 0], isem[q])
            pltpu.async_copy(edges_hbm.at[1, pl.ds(col, CHUNK)], sd.at[q, 1], isem[q])

        def i_wait(i, q):
            col = pl.multiple_of((i * nw + w) * CHUNK, CHUNK)
            pltpu.make_async_copy(
                edges_hbm.at[0, pl.ds(col, CHUNK)], sd.at[q, 0], isem[q]
            ).wait()
            pltpu.make_async_copy(
                edges_hbm.at[1, pl.ds(col, CHUNK)], sd.at[q, 1], isem[q]
            ).wait()

        def g_start(q, b):
            pltpu.async_copy(g_hbm.at[sd.at[q, 0]], rows.at[b], gsem[b])

        def g_wait(q, b):
            pltpu.make_async_copy(g_hbm.at[sd.at[q, 0]], rows.at[b], gsem[b]).wait()

        def s_start(q, b):
            pltpu.async_copy(rows.at[b], acc_shared.at[sd.at[q, 1]], ssem[b], add=True)

        def s_wait(q, b):
            pltpu.make_async_copy(rows.at[b], acc_shared.at[sd.at[q, 1]], ssem[b]).wait()

        @pl.loop(0, CHUNK)
        def _zrows(i):
            for j in range(d // LANES):
                rows[0, i, pl.ds(j * LANES, LANES)] = jnp.zeros((LANES,), jnp.float32)

        @pl.loop(0, drain_chunks)
        def _zacc(kk):
            r = pl.multiple_of(row0 + kk * CHUNK, CHUNK)
            pltpu.sync_copy(rows.at[0], acc_shared.at[pl.ds(r, CHUNK)])

        plsc.subcore_barrier()

        for q in range(2):
            i_start(q, q)
        for q in range(2):
            i_wait(q, q)
            g_start(q, q)

        @pl.loop(0, n_chunks // 4)
        def _pipe(t):
            for kk in range(4):
                i = t * 4 + kk
                q = kk
                qn = (kk + 2) % 4
                bslot = kk % 2

                @pl.when(i + 2 < n_chunks)
                def _pre():
                    i_start(i + 2, qn)

                g_wait(q, bslot)
                s_start(q, bslot)

                @pl.when(i + 2 < n_chunks)
                def _nxt():
                    s_wait(q, bslot)
                    i_wait(i + 2, qn)
                    g_start(qn, bslot)

        s_wait(2, 0)
        s_wait(3, 1)

        plsc.subcore_barrier()

        @pl.loop(0, drain_chunks)
        def _drain(kk):
            r = pl.multiple_of(row0 + kk * CHUNK, CHUNK)
            pltpu.sync_copy(
                acc_shared.at[pl.ds(r, CHUNK)], out_hbm.at[c, pl.ds(r, CHUNK)]
            )

    return k(g, edges)


def _tc_matmul(x, w):
    n, d_in = x.shape
    d_out = w.shape[1]
    blk = 2000

    def body(x_ref, w_ref, h_ref):
        h_ref[...] = jnp.dot(x_ref[...], w_ref[...],
                             preferred_element_type=jnp.float32)

    return pl.pallas_call(
        body,
        grid=(n // blk,),
        in_specs=[
            pl.BlockSpec((blk, d_in), lambda i: (i, 0)),
            pl.BlockSpec((d_in, d_out), lambda i: (0, 0)),
        ],
        out_specs=pl.BlockSpec((blk, d_out), lambda i: (i, 0)),
        out_shape=jax.ShapeDtypeStruct((n, d_out), jnp.float32),
    )(x, w)


def _tc_scale(h, degsum):
    n, d = h.shape
    blk = 2000

    def body(h_ref, deg_ref, g_ref, dis_ref):
        dis = lax.rsqrt(deg_ref[...])
        g_ref[...] = h_ref[...] * dis
        dis_ref[...] = dis

    return pl.pallas_call(
        body,
        grid=(n // blk,),
        in_specs=[
            pl.BlockSpec((blk, d), lambda i: (i, 0)),
            pl.BlockSpec((blk, 1), lambda i: (i, 0)),
        ],
        out_specs=[
            pl.BlockSpec((blk, d), lambda i: (i, 0)),
            pl.BlockSpec((blk, 1), lambda i: (i, 0)),
        ],
        out_shape=[
            jax.ShapeDtypeStruct((n, d), jnp.float32),
            jax.ShapeDtypeStruct((n, 1), jnp.float32),
        ],
    )(h, degsum)


def _tc_combine(acc, g, dis, b2, n):
    d = g.shape[1]
    blk = 2000

    def body(acc_ref, g_ref, dis_ref, b_ref, out_ref):
        total = acc_ref[0] + acc_ref[1] + g_ref[...]
        out_ref[...] = dis_ref[...] * total + b_ref[...]

    return pl.pallas_call(
        body,
        grid=(n // blk,),
        in_specs=[
            pl.BlockSpec((2, blk, d), lambda i: (0, i, 0)),
            pl.BlockSpec((blk, d), lambda i: (i, 0)),
            pl.BlockSpec((blk, 1), lambda i: (i, 0)),
            pl.BlockSpec((1, d), lambda i: (0, 0)),
        ],
        out_specs=pl.BlockSpec((blk, d), lambda i: (i, 0)),
        out_shape=jax.ShapeDtypeStruct((n, d), jnp.float32),
    )(acc, g, dis, b2)


def kernel(x, edge_index, W, b):
    n, d_in = x.shape
    d_out = W.shape[1]
    e = edge_index.shape[1]
    nw = NC * NS

    rows_per_tile = pl.cdiv(n + 1, NS * CHUNK) * CHUNK
    n_pad = NS * rows_per_tile

    e_tile = pl.cdiv(e, nw * 2 * NIDX * CHUNK) * 2 * NIDX * CHUNK
    n_chunks = e_tile // CHUNK
    pad = nw * e_tile - e
    zsrc = jnp.arange(pad, dtype=jnp.int32) % n
    zdst = n + jnp.arange(pad, dtype=jnp.int32) % (n_pad - n)
    edges = jnp.concatenate([edge_index, jnp.stack([zsrc, zdst])], axis=1)

    h = _tc_matmul(x, W)
    degs = _sc_degree(edges, n_pad, rows_per_tile, n_chunks)
    degsum = (degs[0] + degs[1] + 1.0)[:n].reshape(n, 1)
    g, dis = _tc_scale(h, degsum)

    acc = _sc_aggregate(g, edges, n_pad, rows_per_tile, n_chunks)
    out = _tc_combine(acc, g, dis, b.reshape(1, d_out), n)
    return out


# speedup vs baseline: 1.1214x; 1.0196x over previous
import functools

import numpy as np

import jax
import jax.numpy as jnp
from jax import lax
from jax.experimental import pallas as pl
from jax.experimental.pallas import tpu as pltpu
from jax.experimental.pallas import tpu_sc as plsc

NC = 2
NS = 16
LANES = 16
CHUNK = 128
NIDX = 4


def _sc_mesh():
    return plsc.VectorSubcoreMesh(core_axis_name="c", subcore_axis_name="s")


def _sc_degree(edges, n_pad, rows_per_tile, n_chunks):
    nw = NC * NS

    @functools.partial(
        pl.kernel,
        out_type=jax.ShapeDtypeStruct((NC, n_pad), jnp.float32),
        mesh=_sc_mesh(),
        scratch_types=[
            pltpu.VMEM((n_chunks, CHUNK), jnp.int32),
            pltpu.VMEM((CHUNK,), jnp.float32),
            pltpu.VMEM((rows_per_tile,), jnp.float32),
            pltpu.VMEM_SHARED((n_pad,), jnp.float32),
            pltpu.SemaphoreType.DMA,
            pltpu.SemaphoreType.DMA,
        ],
    )
    def k(edges_hbm, out_hbm, idx_all, ones_v, zeros_v, deg_shared, psem, sem):
        c = lax.axis_index("c")
        s = lax.axis_index("s")
        w = c * NS + s
        row0 = pl.multiple_of(s * rows_per_tile, CHUNK)

        @pl.loop(0, n_chunks)
        def _pfire(i):
            col = pl.multiple_of((i * nw + w) * CHUNK, CHUNK)
            pltpu.async_copy(edges_hbm.at[1, pl.ds(col, CHUNK)], idx_all.at[i], psem)

        @pl.loop(0, rows_per_tile // LANES)
        def _zero(i):
            zeros_v[pl.ds(i * LANES, LANES)] = jnp.zeros((LANES,), jnp.float32)

        pltpu.sync_copy(zeros_v, deg_shared.at[pl.ds(row0, rows_per_tile)])
        for j in range(CHUNK // LANES):
            ones_v[pl.ds(j * LANES, LANES)] = jnp.ones((LANES,), jnp.float32)

        @pl.loop(0, n_chunks)
        def _pdrain(i):
            col = pl.multiple_of((i * nw + w) * CHUNK, CHUNK)
            pltpu.make_async_copy(
                edges_hbm.at[1, pl.ds(col, CHUNK)], idx_all.at[i], psem
            ).wait()

        plsc.subcore_barrier()

        @pl.loop(0, n_chunks)
        def _fire(i):
            pltpu.async_copy(ones_v, deg_shared.at[idx_all.at[i]], sem, add=True)

        @pl.loop(0, n_chunks)
        def _drain(i):
            pltpu.make_async_copy(ones_v, deg_shared.at[idx_all.at[i]], sem).wait()

        plsc.subcore_barrier()
        pltpu.sync_copy(
            deg_shared.at[pl.ds(row0, rows_per_tile)],
            out_hbm.at[c, pl.ds(row0, rows_per_tile)],
        )

    return k(edges)


def _sc_aggregate(g, edges, n_pad, rows_per_tile, n_chunks):
    n, d = g.shape
    nw = NC * NS
    drain_chunks = rows_per_tile // CHUNK

    @functools.partial(
        pl.kernel,
        out_type=jax.ShapeDtypeStruct((NC, n_pad, d), jnp.float32),
        mesh=_sc_mesh(),
        scratch_types=[
            pltpu.VMEM((NIDX, 2, CHUNK), jnp.int32),
            pltpu.VMEM((2, CHUNK, d), jnp.float32),
            pltpu.VMEM_SHARED((n_pad, d), jnp.float32),
        ]
        + [pltpu.SemaphoreType.DMA] * (NIDX + 4),
    )
    def k(g_hbm, edges_hbm, out_hbm, sd, rows, acc_shared, *sems):
        isem = sems[:NIDX]
        gsem = sems[NIDX:NIDX + 2]
        ssem = sems[NIDX + 2:]
        c = lax.axis_index("c")
        s = lax.axis_index("s")
        w = c * NS + s
        row0 = pl.multiple_of(s * rows_per_tile, CHUNK)

        def i_start(i, q):
            col = pl.multiple_of((i * nw + w) * CHUNK, CHUNK)
            pltpu.async_copy(edges_hbm.at[0, pl.ds(col, CHUNK)], sd.at[q, 0], isem[q])
            pltpu.async_copy(edges_hbm.at[1, pl.ds(col, CHUNK)], sd.at[q, 1], isem[q])

        def i_wait(i, q):
            col = pl.multiple_of((i * nw + w) * CHUNK, CHUNK)
            pltpu.make_async_copy(
                edges_hbm.at[0, pl.ds(col, CHUNK)], sd.at[q, 0], isem[q]
            ).wait()
            pltpu.make_async_copy(
                edges_hbm.at[1, pl.ds(col, CHUNK)], sd.at[q, 1], isem[q]
            ).wait()

        def g_start(q, b):
            pltpu.async_copy(g_hbm.at[sd.at[q, 0]], rows.at[b], gsem[b])

        def g_wait(q, b):
            pltpu.make_async_copy(g_hbm.at[sd.at[q, 0]], rows.at[b], gsem[b]).wait()

        def s_start(q, b):
            pltpu.async_copy(rows.at[b], acc_shared.at[sd.at[q, 1]], ssem[b], add=True)

        def s_wait(q, b):
            pltpu.make_async_copy(rows.at[b], acc_shared.at[sd.at[q, 1]], ssem[b]).wait()

        @pl.loop(0, CHUNK)
        def _zrows(i):
            for j in range(d // LANES):
                rows[0, i, pl.ds(j * LANES, LANES)] = jnp.zeros((LANES,), jnp.float32)

        @pl.loop(0, drain_chunks)
        def _zacc(kk):
            r = pl.multiple_of(row0 + kk * CHUNK, CHUNK)
            pltpu.sync_copy(rows.at[0], acc_shared.at[pl.ds(r, CHUNK)])

        plsc.subcore_barrier()

        for q in range(2):
            i_start(q, q)
        for q in range(2):
            i_wait(q, q)
            g_start(q, q)

        @pl.loop(0, n_chunks // 4)
        def _pipe(t):
            for kk in range(4):
                i = t * 4 + kk
                q = kk
                qn = (kk + 2) % 4
                bslot = kk % 2

                @pl.when(i + 2 < n_chunks)
                def _pre():
                    i_start(i + 2, qn)

                g_wait(q, bslot)
                s_start(q, bslot)

                @pl.when(i + 2 < n_chunks)
                def _nxt():
                    s_wait(q, bslot)
                    i_wait(i + 2, qn)
                    g_start(qn, bslot)

        s_wait(2, 0)
        s_wait(3, 1)

        plsc.subcore_barrier()

        @pl.loop(0, drain_chunks)
        def _drain(kk):
            r = pl.multiple_of(row0 + kk * CHUNK, CHUNK)
            pltpu.sync_copy(
                acc_shared.at[pl.ds(r, CHUNK)], out_hbm.at[c, pl.ds(r, CHUNK)]
            )

    return k(g, edges)


def _tc_matmul(x, w):
    n, d_in = x.shape
    d_out = w.shape[1]
    blk = 2000

    def body(x_ref, w_ref, h_ref):
        h_ref[...] = jnp.dot(x_ref[...], w_ref[...],
                             preferred_element_type=jnp.float32)

    return pl.pallas_call(
        body,
        grid=(n // blk,),
        in_specs=[
            pl.BlockSpec((blk, d_in), lambda i: (i, 0)),
            pl.BlockSpec((d_in, d_out), lambda i: (0, 0)),
        ],
        out_specs=pl.BlockSpec((blk, d_out), lambda i: (i, 0)),
        out_shape=jax.ShapeDtypeStruct((n, d_out), jnp.float32),
    )(x, w)


def _tc_scale(h, degsum):
    n, d = h.shape

    def body(h_ref, deg_ref, g_ref, dis_ref):
        dis = lax.rsqrt(deg_ref[...])
        g_ref[...] = h_ref[...] * dis
        dis_ref[...] = dis

    return pl.pallas_call(
        body,
        grid=(1,),
        in_specs=[
            pl.BlockSpec((n, d), lambda i: (0, 0)),
            pl.BlockSpec((n, 1), lambda i: (0, 0)),
        ],
        out_specs=[
            pl.BlockSpec((n, d), lambda i: (0, 0)),
            pl.BlockSpec((n, 1), lambda i: (0, 0)),
        ],
        out_shape=[
            jax.ShapeDtypeStruct((n, d), jnp.float32),
            jax.ShapeDtypeStruct((n, 1), jnp.float32),
        ],
    )(h, degsum)


def _tc_combine(acc, g, dis, b2, n):
    d = g.shape[1]
    blk = 2000

    def body(acc_ref, g_ref, dis_ref, b_ref, out_ref):
        total = acc_ref[0] + acc_ref[1] + g_ref[...]
        out_ref[...] = dis_ref[...] * total + b_ref[...]

    return pl.pallas_call(
        body,
        grid=(n // blk,),
        in_specs=[
            pl.BlockSpec((2, blk, d), lambda i: (0, i, 0)),
            pl.BlockSpec((blk, d), lambda i: (i, 0)),
            pl.BlockSpec((blk, 1), lambda i: (i, 0)),
            pl.BlockSpec((1, d), lambda i: (0, 0)),
        ],
        out_specs=pl.BlockSpec((blk, d), lambda i: (i, 0)),
        out_shape=jax.ShapeDtypeStruct((n, d), jnp.float32),
    )(acc, g, dis, b2)


def kernel(x, edge_index, W, b):
    n, d_in = x.shape
    d_out = W.shape[1]
    e = edge_index.shape[1]
    nw = NC * NS

    rows_per_tile = pl.cdiv(n + 1, NS * CHUNK) * CHUNK
    n_pad = NS * rows_per_tile

    e_tile = pl.cdiv(e, nw * 2 * NIDX * CHUNK) * 2 * NIDX * CHUNK
    n_chunks = e_tile // CHUNK
    pad = nw * e_tile - e
    jpad = np.arange(pad)
    zpad = np.stack([jpad % n, n + jpad % (n_pad - n)]).astype(np.int32)
    edges = jnp.concatenate([edge_index, jnp.asarray(zpad)], axis=1)

    h = _tc_matmul(x, W)
    degs = _sc_degree(edges, n_pad, rows_per_tile, n_chunks)
    degsum = (degs[0] + degs[1] + 1.0).reshape(n_pad, 1)
    g, dis = _tc_scale(h, degsum)

    acc = _sc_aggregate(g, edges, n_pad, rows_per_tile, n_chunks)
    out = _tc_combine(acc, g, dis, b.reshape(1, d_out), n)
    return out
